# initial kernel scaffold (unmeasured)
import jax
import jax.numpy as jnp
from jax import lax
from jax.experimental import pallas as pl
from jax.experimental.pallas import tpu as pltpu

N_DEV = 8


def _ring_to_pos(r):
    return jnp.where(r < 4, r, 11 - r)


def kernel(x, w_mat):
    m, k_per = x.shape
    _, n = w_mat.shape
    mc = m // N_DEV

    x = x.astype(jnp.bfloat16)
    w_mat = w_mat.astype(jnp.bfloat16)

    def body(x_ref, w_ref, out_ref, acc_ref, comm_ref, send_ref,
             send_sems, recv_sems, copy_sem, credit_sem):
        my = lax.axis_index("i")
        q = _ring_to_pos(my)
        nxt = _ring_to_pos(jnp.mod(q + 1, N_DEV))
        prv = _ring_to_pos(jnp.mod(q - 1, N_DEV))

        barrier = pltpu.get_barrier_semaphore()
        for nbr in (prv, nxt):
            pl.semaphore_signal(barrier, inc=1, device_id=(nbr,),
                                device_id_type=pl.DeviceIdType.MESH)
        pl.semaphore_wait(barrier, 2)

        def partial(c):
            return jnp.dot(x_ref[pl.ds(c * mc, mc), :], w_ref[...],
                           preferred_element_type=jnp.float32)

        def signal_credit():
            pl.semaphore_signal(credit_sem, inc=1, device_id=(prv,),
                                device_id_type=pl.DeviceIdType.MESH)


        acc_ref[...] = partial(jnp.mod(q - 1, N_DEV))
        for s in range(N_DEV - 1):
            slot = s % 2
            if s >= 2:
                pl.semaphore_wait(credit_sem, 1)
            send_ref[slot] = acc_ref[...].astype(jnp.bfloat16)
            rdma = pltpu.make_async_remote_copy(
                src_ref=send_ref.at[slot],
                dst_ref=comm_ref.at[slot],
                send_sem=send_sems.at[slot],
                recv_sem=recv_sems.at[slot],
                device_id=(nxt,),
                device_id_type=pl.DeviceIdType.MESH,
            )
            rdma.start()
            acc_ref[...] = partial(jnp.mod(q - 2 - s, N_DEV))
            rdma.wait()
            acc_ref[...] = acc_ref[...] + comm_ref[slot].astype(jnp.float32)
            if s >= 1:
                signal_credit()

        y = acc_ref[...]
        y_bf16 = (y * jax.nn.sigmoid(y)).astype(jnp.bfloat16)

        for t in range(N_DEV - 1):
            s = (N_DEV - 1) + t
            slot = s % 2
            pl.semaphore_wait(credit_sem, 1)
            if t == 0:
                send_ref[slot] = y_bf16
                src = send_ref.at[slot]
            else:
                src = comm_ref.at[(s - 1) % 2]
            rdma = pltpu.make_async_remote_copy(
                src_ref=src,
                dst_ref=comm_ref.at[slot],
                send_sem=send_sems.at[slot],
                recv_sem=recv_sems.at[slot],
                device_id=(nxt,),
                device_id_type=pl.DeviceIdType.MESH,
            )
            rdma.start()
            if t == 0:
                own = pltpu.make_async_copy(
                    send_ref.at[slot], out_ref.at[pl.ds(q * mc, mc), :],
                    copy_sem)
                own.start()
                own.wait()
            rdma.wait()
            got = jnp.mod(q - 1 - t, N_DEV)
            cp = pltpu.make_async_copy(
                comm_ref.at[slot], out_ref.at[pl.ds(got * mc, mc), :],
                copy_sem)
            cp.start()
            cp.wait()
            if s <= 12:
                signal_credit()

    return pl.pallas_call(
        body,
        out_shape=jax.ShapeDtypeStruct((m, n), jnp.bfloat16),
        in_specs=[
            pl.BlockSpec(memory_space=pltpu.VMEM),
            pl.BlockSpec(memory_space=pltpu.VMEM),
        ],
        out_specs=pl.BlockSpec(memory_space=pltpu.ANY),
        scratch_shapes=[
            pltpu.VMEM((mc, n), jnp.float32),
            pltpu.VMEM((2, mc, n), jnp.bfloat16),
            pltpu.VMEM((2, mc, n), jnp.bfloat16),
            pltpu.SemaphoreType.DMA((2,)),
            pltpu.SemaphoreType.DMA((2,)),
            pltpu.SemaphoreType.DMA,
            pltpu.SemaphoreType.REGULAR,
        ],
        compiler_params=pltpu.CompilerParams(
            collective_id=0,
            vmem_limit_bytes=128 * 1024 * 1024,
        ),
    )(x, w_mat)


# baseline (device time: 1400168 ns/iter reference)
import jax
import jax.numpy as jnp
from jax import lax
from jax.experimental import pallas as pl
from jax.experimental.pallas import tpu as pltpu

N_DEV = 8


def _ring_to_pos(r):
    return jnp.where(r < 4, r, 11 - r)


def kernel(x, w_mat):
    m, k_per = x.shape
    _, n = w_mat.shape
    mc = m // N_DEV

    x = x.astype(jnp.bfloat16)
    w_mat = w_mat.astype(jnp.bfloat16)

    def body(x_ref, w_ref, out_ref, acc_ref, comm_ref,
             send_sem, recv_sem, copy_sem, credit_sem):
        my = lax.axis_index("i")
        q = _ring_to_pos(my)
        nxt = _ring_to_pos(jnp.mod(q + 1, N_DEV))
        prv = _ring_to_pos(jnp.mod(q - 1, N_DEV))

        barrier = pltpu.get_barrier_semaphore()
        for nbr in (prv, nxt):
            pl.semaphore_signal(barrier, inc=1, device_id=(nbr,),
                                device_id_type=pl.DeviceIdType.MESH)
        pl.semaphore_wait(barrier, 2)

        def partial(c):
            return jnp.dot(x_ref[pl.ds(c * mc, mc), :], w_ref[...],
                           preferred_element_type=jnp.float32
                           ).astype(jnp.bfloat16)

        def send_to_next(src):
            return pltpu.make_async_remote_copy(
                src_ref=src,
                dst_ref=comm_ref,
                send_sem=send_sem,
                recv_sem=recv_sem,
                device_id=(nxt,),
                device_id_type=pl.DeviceIdType.MESH,
            )

        def signal_credit():
            pl.semaphore_signal(credit_sem, inc=1, device_id=(prv,),
                                device_id_type=pl.DeviceIdType.MESH)

        acc_ref[0] = partial(jnp.mod(q - 1, N_DEV))
        for s in range(N_DEV - 1):
            slot = s % 2
            if s >= 1:
                pl.semaphore_wait(credit_sem, 1)
            rdma = send_to_next(acc_ref.at[slot])
            rdma.start()
            acc_ref[1 - slot] = partial(jnp.mod(q - 2 - s, N_DEV))
            rdma.wait()
            acc_ref[1 - slot] = acc_ref[1 - slot] + comm_ref[...]
            signal_credit()

        y = acc_ref[1].astype(jnp.float32)
        acc_ref[1] = (y * jax.nn.sigmoid(y)).astype(jnp.bfloat16)
        own = pltpu.make_async_copy(
            acc_ref.at[1], out_ref.at[pl.ds(q * mc, mc), :], copy_sem)
        own.start()
        own.wait()

        for t in range(N_DEV - 1):
            s = (N_DEV - 1) + t
            pl.semaphore_wait(credit_sem, 1)
            c_fwd = jnp.mod(q - t, N_DEV)
            rdma = send_to_next(out_ref.at[pl.ds(c_fwd * mc, mc), :])
            rdma.start()
            rdma.wait()
            c_got = jnp.mod(q - 1 - t, N_DEV)
            cp = pltpu.make_async_copy(
                comm_ref, out_ref.at[pl.ds(c_got * mc, mc), :], copy_sem)
            cp.start()
            cp.wait()
            if s <= 12:
                signal_credit()

    return pl.pallas_call(
        body,
        out_shape=jax.ShapeDtypeStruct((m, n), jnp.bfloat16),
        in_specs=[
            pl.BlockSpec(memory_space=pltpu.VMEM),
            pl.BlockSpec(memory_space=pltpu.VMEM),
        ],
        out_specs=pl.BlockSpec(memory_space=pl.ANY),
        scratch_shapes=[
            pltpu.VMEM((2, mc, n), jnp.bfloat16),
            pltpu.VMEM((mc, n), jnp.bfloat16),
            pltpu.SemaphoreType.DMA,
            pltpu.SemaphoreType.DMA,
            pltpu.SemaphoreType.DMA,
            pltpu.SemaphoreType.REGULAR,
        ],
        compiler_params=pltpu.CompilerParams(
            collective_id=0,
            vmem_limit_bytes=64 * 1024 * 1024,
        ),
    )(x, w_mat)


# device time: 798178 ns/iter; 1.7542x vs baseline; 1.7542x over previous
import jax
import jax.numpy as jnp
from jax import lax
from jax.experimental import pallas as pl
from jax.experimental.pallas import tpu as pltpu

N_DEV = 8


def _ring_to_pos(r):
    return jnp.where(r < 4, r, 11 - r)


def kernel(x, w_mat):
    m, k_per = x.shape
    _, n = w_mat.shape
    mc = m // N_DEV
    h = mc // 2

    x = x.astype(jnp.bfloat16)
    w_mat = w_mat.astype(jnp.bfloat16)

    def body(x_ref, w_ref, out_ref, acc_ref, comm_cw_ref, comm_ccw_ref,
             send_sem_cw, recv_sem_cw, send_sem_ccw, recv_sem_ccw,
             copy_sem, credit_cw, credit_ccw):
        my = lax.axis_index("i")
        q = _ring_to_pos(my)
        nxt = _ring_to_pos(jnp.mod(q + 1, N_DEV))
        prv = _ring_to_pos(jnp.mod(q - 1, N_DEV))

        barrier = pltpu.get_barrier_semaphore()
        for nbr in (prv, nxt):
            pl.semaphore_signal(barrier, inc=1, device_id=(nbr,),
                                device_id_type=pl.DeviceIdType.MESH)
        pl.semaphore_wait(barrier, 2)

        def partial_top(c):
            return jnp.dot(x_ref[pl.ds(c * mc, h), :], w_ref[...],
                           preferred_element_type=jnp.float32
                           ).astype(jnp.bfloat16)

        def partial_bot(c):
            return jnp.dot(x_ref[pl.ds(c * mc + h, h), :], w_ref[...],
                           preferred_element_type=jnp.float32
                           ).astype(jnp.bfloat16)

        def send_cw(src):
            return pltpu.make_async_remote_copy(
                src_ref=src, dst_ref=comm_cw_ref,
                send_sem=send_sem_cw, recv_sem=recv_sem_cw,
                device_id=(nxt,), device_id_type=pl.DeviceIdType.MESH)

        def send_ccw(src):
            return pltpu.make_async_remote_copy(
                src_ref=src, dst_ref=comm_ccw_ref,
                send_sem=send_sem_ccw, recv_sem=recv_sem_ccw,
                device_id=(prv,), device_id_type=pl.DeviceIdType.MESH)

        def wait_credits():
            pl.semaphore_wait(credit_cw, 1)
            pl.semaphore_wait(credit_ccw, 1)

        def signal_credits():
            pl.semaphore_signal(credit_cw, inc=1, device_id=(prv,),
                                device_id_type=pl.DeviceIdType.MESH)
            pl.semaphore_signal(credit_ccw, inc=1, device_id=(nxt,),
                                device_id_type=pl.DeviceIdType.MESH)

        acc_ref[:h] = partial_top(jnp.mod(q - 1, N_DEV))
        acc_ref[h:] = partial_bot(jnp.mod(q + 1, N_DEV))

        def rs_step(s, carry):
            @pl.when(s >= 1)
            def _():
                wait_credits()
            rdma_cw = send_cw(acc_ref.at[pl.ds(0, h), :])
            rdma_ccw = send_ccw(acc_ref.at[pl.ds(h, h), :])
            rdma_cw.start()
            rdma_ccw.start()
            rdma_cw.wait_send()
            rdma_ccw.wait_send()
            acc_ref[:h] = partial_top(jnp.mod(q - 2 - s, N_DEV))
            acc_ref[h:] = partial_bot(jnp.mod(q + 2 + s, N_DEV))
            rdma_cw.wait_recv()
            rdma_ccw.wait_recv()
            acc_ref[:h] = acc_ref[:h] + comm_cw_ref[...]
            acc_ref[h:] = acc_ref[h:] + comm_ccw_ref[...]
            signal_credits()
            return carry

        lax.fori_loop(0, N_DEV - 1, rs_step, 0)

        y = acc_ref[...].astype(jnp.float32)
        acc_ref[...] = (y * jax.nn.sigmoid(y)).astype(jnp.bfloat16)
        own = pltpu.make_async_copy(
            acc_ref, out_ref.at[pl.ds(q * mc, mc), :], copy_sem)
        own.start()
        own.wait()

        def ag_step(t, carry):
            wait_credits()
            c_cw = jnp.mod(q - t, N_DEV)
            c_ccw = jnp.mod(q + t, N_DEV)
            rdma_cw = send_cw(out_ref.at[pl.ds(c_cw * mc, h), :])
            rdma_ccw = send_ccw(out_ref.at[pl.ds(c_ccw * mc + h, h), :])
            rdma_cw.start()
            rdma_ccw.start()
            rdma_cw.wait()
            rdma_ccw.wait()
            g_cw = jnp.mod(q - 1 - t, N_DEV)
            g_ccw = jnp.mod(q + 1 + t, N_DEV)
            cp_cw = pltpu.make_async_copy(
                comm_cw_ref, out_ref.at[pl.ds(g_cw * mc, h), :], copy_sem)
            cp_cw.start()
            cp_ccw = pltpu.make_async_copy(
                comm_ccw_ref, out_ref.at[pl.ds(g_ccw * mc + h, h), :],
                copy_sem)
            cp_ccw.start()
            cp_cw.wait()
            cp_ccw.wait()

            @pl.when(t < N_DEV - 2)
            def _():
                signal_credits()

            return carry

        lax.fori_loop(0, N_DEV - 1, ag_step, 0)

    return pl.pallas_call(
        body,
        out_shape=jax.ShapeDtypeStruct((m, n), jnp.bfloat16),
        in_specs=[
            pl.BlockSpec(memory_space=pltpu.VMEM),
            pl.BlockSpec(memory_space=pltpu.VMEM),
        ],
        out_specs=pl.BlockSpec(memory_space=pl.ANY),
        scratch_shapes=[
            pltpu.VMEM((mc, n), jnp.bfloat16),
            pltpu.VMEM((h, n), jnp.bfloat16),
            pltpu.VMEM((h, n), jnp.bfloat16),
            pltpu.SemaphoreType.DMA,
            pltpu.SemaphoreType.DMA,
            pltpu.SemaphoreType.DMA,
            pltpu.SemaphoreType.DMA,
            pltpu.SemaphoreType.DMA,
            pltpu.SemaphoreType.REGULAR,
            pltpu.SemaphoreType.REGULAR,
        ],
        compiler_params=pltpu.CompilerParams(
            collective_id=0,
            vmem_limit_bytes=64 * 1024 * 1024,
        ),
    )(x, w_mat)


# device time: 775727 ns/iter; 1.8050x vs baseline; 1.0289x over previous
import jax
import jax.numpy as jnp
from jax import lax
from jax.experimental import pallas as pl
from jax.experimental.pallas import tpu as pltpu

N_DEV = 8


def _ring_to_pos(r):
    return jnp.where(r < 4, r, 11 - r)


def kernel(x, w_mat):
    m, k_per = x.shape
    _, n = w_mat.shape
    mc = m // N_DEV
    h = mc // 2

    x = x.astype(jnp.bfloat16)
    w_mat = w_mat.astype(jnp.bfloat16)

    def body(x_ref, w_ref, out_ref, acc_ref, stage_cw_ref, stage_ccw_ref,
             comm_cw_ref, comm_ccw_ref,
             send_sem_cw, recv_sem_cw, send_sem_ccw, recv_sem_ccw,
             copy_sem, credit_cw, credit_ccw):
        my = lax.axis_index("i")
        q = _ring_to_pos(my)
        nxt = _ring_to_pos(jnp.mod(q + 1, N_DEV))
        prv = _ring_to_pos(jnp.mod(q - 1, N_DEV))

        barrier = pltpu.get_barrier_semaphore()
        for nbr in (prv, nxt):
            pl.semaphore_signal(barrier, inc=1, device_id=(nbr,),
                                device_id_type=pl.DeviceIdType.MESH)
        pl.semaphore_wait(barrier, 2)

        def partial_top(c):
            return jnp.dot(x_ref[pl.ds(c * mc, h), :], w_ref[...],
                           preferred_element_type=jnp.float32
                           ).astype(jnp.bfloat16)

        def partial_bot(c):
            return jnp.dot(x_ref[pl.ds(c * mc + h, h), :], w_ref[...],
                           preferred_element_type=jnp.float32
                           ).astype(jnp.bfloat16)

        def send_cw(src):
            return pltpu.make_async_remote_copy(
                src_ref=src, dst_ref=comm_cw_ref,
                send_sem=send_sem_cw, recv_sem=recv_sem_cw,
                device_id=(nxt,), device_id_type=pl.DeviceIdType.MESH)

        def send_ccw(src):
            return pltpu.make_async_remote_copy(
                src_ref=src, dst_ref=comm_ccw_ref,
                send_sem=send_sem_ccw, recv_sem=recv_sem_ccw,
                device_id=(prv,), device_id_type=pl.DeviceIdType.MESH)

        def wait_credits():
            pl.semaphore_wait(credit_cw, 1)
            pl.semaphore_wait(credit_ccw, 1)

        def signal_credits():
            pl.semaphore_signal(credit_cw, inc=1, device_id=(prv,),
                                device_id_type=pl.DeviceIdType.MESH)
            pl.semaphore_signal(credit_ccw, inc=1, device_id=(nxt,),
                                device_id_type=pl.DeviceIdType.MESH)

        acc_ref[:h] = partial_top(jnp.mod(q - 1, N_DEV))
        acc_ref[h:] = partial_bot(jnp.mod(q + 1, N_DEV))

        def rs_step(s, carry):
            @pl.when(s >= 1)
            def _():
                send_cw(stage_cw_ref).wait_send()
                send_ccw(stage_ccw_ref).wait_send()
                wait_credits()
            stage_cw_ref[...] = acc_ref[:h]
            stage_ccw_ref[...] = acc_ref[h:]
            rdma_cw = send_cw(stage_cw_ref)
            rdma_ccw = send_ccw(stage_ccw_ref)
            rdma_cw.start()
            rdma_ccw.start()
            acc_ref[:h] = partial_top(jnp.mod(q - 2 - s, N_DEV))
            acc_ref[h:] = partial_bot(jnp.mod(q + 2 + s, N_DEV))
            rdma_cw.wait_recv()
            rdma_ccw.wait_recv()
            acc_ref[:h] = acc_ref[:h] + comm_cw_ref[...]
            acc_ref[h:] = acc_ref[h:] + comm_ccw_ref[...]
            signal_credits()
            return carry

        lax.fori_loop(0, N_DEV - 1, rs_step, 0)
        send_cw(stage_cw_ref).wait_send()
        send_ccw(stage_ccw_ref).wait_send()

        y = acc_ref[...].astype(jnp.float32)
        acc_ref[...] = (y * jax.nn.sigmoid(y)).astype(jnp.bfloat16)
        own = pltpu.make_async_copy(
            acc_ref, out_ref.at[pl.ds(q * mc, mc), :], copy_sem)
        own.start()
        own.wait()

        def ag_step(t, carry):
            wait_credits()
            c_cw = jnp.mod(q - t, N_DEV)
            c_ccw = jnp.mod(q + t, N_DEV)
            rdma_cw = send_cw(out_ref.at[pl.ds(c_cw * mc, h), :])
            rdma_ccw = send_ccw(out_ref.at[pl.ds(c_ccw * mc + h, h), :])
            rdma_cw.start()
            rdma_ccw.start()
            rdma_cw.wait()
            rdma_ccw.wait()
            g_cw = jnp.mod(q - 1 - t, N_DEV)
            g_ccw = jnp.mod(q + 1 + t, N_DEV)
            cp_cw = pltpu.make_async_copy(
                comm_cw_ref, out_ref.at[pl.ds(g_cw * mc, h), :], copy_sem)
            cp_cw.start()
            cp_ccw = pltpu.make_async_copy(
                comm_ccw_ref, out_ref.at[pl.ds(g_ccw * mc + h, h), :],
                copy_sem)
            cp_ccw.start()
            cp_cw.wait()
            cp_ccw.wait()

            @pl.when(t < N_DEV - 2)
            def _():
                signal_credits()

            return carry

        lax.fori_loop(0, N_DEV - 1, ag_step, 0)

    return pl.pallas_call(
        body,
        out_shape=jax.ShapeDtypeStruct((m, n), jnp.bfloat16),
        in_specs=[
            pl.BlockSpec(memory_space=pltpu.VMEM),
            pl.BlockSpec(memory_space=pltpu.VMEM),
        ],
        out_specs=pl.BlockSpec(memory_space=pl.ANY),
        scratch_shapes=[
            pltpu.VMEM((mc, n), jnp.bfloat16),
            pltpu.VMEM((h, n), jnp.bfloat16),
            pltpu.VMEM((h, n), jnp.bfloat16),
            pltpu.VMEM((h, n), jnp.bfloat16),
            pltpu.VMEM((h, n), jnp.bfloat16),
            pltpu.SemaphoreType.DMA,
            pltpu.SemaphoreType.DMA,
            pltpu.SemaphoreType.DMA,
            pltpu.SemaphoreType.DMA,
            pltpu.SemaphoreType.DMA,
            pltpu.SemaphoreType.REGULAR,
            pltpu.SemaphoreType.REGULAR,
        ],
        compiler_params=pltpu.CompilerParams(
            collective_id=0,
            vmem_limit_bytes=64 * 1024 * 1024,
        ),
    )(x, w_mat)


# device time: 774039 ns/iter; 1.8089x vs baseline; 1.0022x over previous
import jax
import jax.numpy as jnp
from jax import lax
from jax.experimental import pallas as pl
from jax.experimental.pallas import tpu as pltpu

N_DEV = 8


def _ring_to_pos(r):
    return jnp.where(r < 4, r, 11 - r)


def kernel(x, w_mat):
    m, k_per = x.shape
    _, n = w_mat.shape
    mc = m // N_DEV
    h = mc // 2

    x = x.astype(jnp.bfloat16)
    w_mat = w_mat.astype(jnp.bfloat16)

    def body(x_ref, w_ref, out_ref, acc_ref, stage_cw_ref, stage_ccw_ref,
             comm_cw_ref, comm_ccw_ref,
             send_sem_cw, recv_sem_cw, send_sem_ccw, recv_sem_ccw,
             copy_sem, credit_cw, credit_ccw):
        my = lax.axis_index("i")
        q = _ring_to_pos(my)
        nxt = _ring_to_pos(jnp.mod(q + 1, N_DEV))
        prv = _ring_to_pos(jnp.mod(q - 1, N_DEV))

        barrier = pltpu.get_barrier_semaphore()
        for nbr in (prv, nxt):
            pl.semaphore_signal(barrier, inc=1, device_id=(nbr,),
                                device_id_type=pl.DeviceIdType.MESH)
        pl.semaphore_wait(barrier, 2)

        def partial_top(c):
            return jnp.dot(x_ref[pl.ds(c * mc, h), :], w_ref[...],
                           preferred_element_type=jnp.float32
                           ).astype(jnp.bfloat16)

        def partial_bot(c):
            return jnp.dot(x_ref[pl.ds(c * mc + h, h), :], w_ref[...],
                           preferred_element_type=jnp.float32
                           ).astype(jnp.bfloat16)

        def rdma_cw(src, dst):
            return pltpu.make_async_remote_copy(
                src_ref=src, dst_ref=dst,
                send_sem=send_sem_cw, recv_sem=recv_sem_cw,
                device_id=(nxt,), device_id_type=pl.DeviceIdType.MESH)

        def rdma_ccw(src, dst):
            return pltpu.make_async_remote_copy(
                src_ref=src, dst_ref=dst,
                send_sem=send_sem_ccw, recv_sem=recv_sem_ccw,
                device_id=(prv,), device_id_type=pl.DeviceIdType.MESH)

        def wait_credits():
            pl.semaphore_wait(credit_cw, 1)
            pl.semaphore_wait(credit_ccw, 1)

        def signal_credits():
            pl.semaphore_signal(credit_cw, inc=1, device_id=(prv,),
                                device_id_type=pl.DeviceIdType.MESH)
            pl.semaphore_signal(credit_ccw, inc=1, device_id=(nxt,),
                                device_id_type=pl.DeviceIdType.MESH)

        stage_cw_ref[...] = partial_top(jnp.mod(q - 1, N_DEV))
        stage_ccw_ref[...] = partial_bot(jnp.mod(q + 1, N_DEV))

        def rs_step(s, carry):
            @pl.when(s >= 1)
            def _():
                wait_credits()
            r_cw = rdma_cw(stage_cw_ref, comm_cw_ref)
            r_ccw = rdma_ccw(stage_ccw_ref, comm_ccw_ref)
            r_cw.start()
            r_ccw.start()
            acc_ref[:h] = partial_top(jnp.mod(q - 2 - s, N_DEV))
            acc_ref[h:] = partial_bot(jnp.mod(q + 2 + s, N_DEV))
            r_cw.wait_recv()
            r_ccw.wait_recv()
            r_cw.wait_send()
            r_ccw.wait_send()
            stage_cw_ref[...] = acc_ref[:h] + comm_cw_ref[...]
            stage_ccw_ref[...] = acc_ref[h:] + comm_ccw_ref[...]
            signal_credits()
            return carry

        lax.fori_loop(0, N_DEV - 1, rs_step, 0)

        y_top = stage_cw_ref[...].astype(jnp.float32)
        stage_cw_ref[...] = (y_top * jax.nn.sigmoid(y_top)).astype(jnp.bfloat16)
        y_bot = stage_ccw_ref[...].astype(jnp.float32)
        stage_ccw_ref[...] = (y_bot * jax.nn.sigmoid(y_bot)).astype(jnp.bfloat16)
        own_cw = pltpu.make_async_copy(
            stage_cw_ref, out_ref.at[pl.ds(q * mc, h), :], copy_sem)
        own_ccw = pltpu.make_async_copy(
            stage_ccw_ref, out_ref.at[pl.ds(q * mc + h, h), :], copy_sem)
        own_cw.start()
        own_ccw.start()
        own_cw.wait()
        own_ccw.wait()

        def ag_step(t, carry):
            wait_credits()
            c_cw = jnp.mod(q - t, N_DEV)
            c_ccw = jnp.mod(q + t, N_DEV)
            r_cw = rdma_cw(out_ref.at[pl.ds(c_cw * mc, h), :], comm_cw_ref)
            r_ccw = rdma_ccw(out_ref.at[pl.ds(c_ccw * mc + h, h), :],
                             comm_ccw_ref)
            r_cw.start()
            r_ccw.start()
            r_cw.wait()
            r_ccw.wait()
            g_cw = jnp.mod(q - 1 - t, N_DEV)
            g_ccw = jnp.mod(q + 1 + t, N_DEV)
            cp_cw = pltpu.make_async_copy(
                comm_cw_ref, out_ref.at[pl.ds(g_cw * mc, h), :], copy_sem)
            cp_cw.start()
            cp_ccw = pltpu.make_async_copy(
                comm_ccw_ref, out_ref.at[pl.ds(g_ccw * mc + h, h), :],
                copy_sem)
            cp_ccw.start()
            cp_cw.wait()
            cp_ccw.wait()

            @pl.when(t < N_DEV - 2)
            def _():
                signal_credits()

            return carry

        lax.fori_loop(0, N_DEV - 1, ag_step, 0)

    return pl.pallas_call(
        body,
        out_shape=jax.ShapeDtypeStruct((m, n), jnp.bfloat16),
        in_specs=[
            pl.BlockSpec(memory_space=pltpu.VMEM),
            pl.BlockSpec(memory_space=pltpu.VMEM),
        ],
        out_specs=pl.BlockSpec(memory_space=pl.ANY),
        scratch_shapes=[
            pltpu.VMEM((mc, n), jnp.bfloat16),
            pltpu.VMEM((h, n), jnp.bfloat16),
            pltpu.VMEM((h, n), jnp.bfloat16),
            pltpu.VMEM((h, n), jnp.bfloat16),
            pltpu.VMEM((h, n), jnp.bfloat16),
            pltpu.SemaphoreType.DMA,
            pltpu.SemaphoreType.DMA,
            pltpu.SemaphoreType.DMA,
            pltpu.SemaphoreType.DMA,
            pltpu.SemaphoreType.DMA,
            pltpu.SemaphoreType.REGULAR,
            pltpu.SemaphoreType.REGULAR,
        ],
        compiler_params=pltpu.CompilerParams(
            collective_id=0,
            vmem_limit_bytes=64 * 1024 * 1024,
        ),
    )(x, w_mat)


# device time: 722576 ns/iter; 1.9377x vs baseline; 1.0712x over previous
import jax
import jax.numpy as jnp
from jax import lax
from jax.experimental import pallas as pl
from jax.experimental.pallas import tpu as pltpu

N_DEV = 8
NSUB = 2


def _ring_to_pos(r):
    return jnp.where(r < 4, r, 11 - r)


def kernel(x, w_mat):
    m, k_per = x.shape
    _, n = w_mat.shape
    mc = m // N_DEV
    h = mc // 2
    nh = n // NSUB

    x = x.astype(jnp.bfloat16)
    w_mat = w_mat.astype(jnp.bfloat16)

    def body(x_ref, w_ref, out_ref, acc_ref, stage_cw_ref, stage_ccw_ref,
             comm_cw_ref, comm_ccw_ref,
             send_sems_cw, recv_sems_cw, send_sems_ccw, recv_sems_ccw,
             copy_sem, credit_cw, credit_ccw):
        my = lax.axis_index("i")
        q = _ring_to_pos(my)
        nxt = _ring_to_pos(jnp.mod(q + 1, N_DEV))
        prv = _ring_to_pos(jnp.mod(q - 1, N_DEV))

        barrier = pltpu.get_barrier_semaphore()
        for nbr in (prv, nxt):
            pl.semaphore_signal(barrier, inc=1, device_id=(nbr,),
                                device_id_type=pl.DeviceIdType.MESH)
        pl.semaphore_wait(barrier, 2)

        def partial_top(c):
            return jnp.dot(x_ref[pl.ds(c * mc, h), :], w_ref[...],
                           preferred_element_type=jnp.float32
                           ).astype(jnp.bfloat16)

        def partial_bot(c):
            return jnp.dot(x_ref[pl.ds(c * mc + h, h), :], w_ref[...],
                           preferred_element_type=jnp.float32
                           ).astype(jnp.bfloat16)

        def rs_rdma(v, cw):
            if cw:
                return pltpu.make_async_remote_copy(
                    src_ref=stage_cw_ref.at[v], dst_ref=comm_cw_ref.at[v],
                    send_sem=send_sems_cw.at[v], recv_sem=recv_sems_cw.at[v],
                    device_id=(nxt,), device_id_type=pl.DeviceIdType.MESH)
            return pltpu.make_async_remote_copy(
                src_ref=stage_ccw_ref.at[v], dst_ref=comm_ccw_ref.at[v],
                send_sem=send_sems_ccw.at[v], recv_sem=recv_sems_ccw.at[v],
                device_id=(prv,), device_id_type=pl.DeviceIdType.MESH)

        pt = partial_top(jnp.mod(q - 1, N_DEV))
        stage_cw_ref[0] = pt[:, :nh]
        stage_cw_ref[1] = pt[:, nh:]
        pb = partial_bot(jnp.mod(q + 1, N_DEV))
        stage_ccw_ref[0] = pb[:, :nh]
        stage_ccw_ref[1] = pb[:, nh:]
        for v in range(NSUB):
            rs_rdma(v, True).start()
            rs_rdma(v, False).start()

        def rs_step(s, carry):
            acc_ref[:h] = partial_top(jnp.mod(q - 1 - s, N_DEV))
            acc_ref[h:] = partial_bot(jnp.mod(q + 1 + s, N_DEV))
            for v in range(NSUB):
                col = pl.ds(v * nh, nh)
                d_cw = rs_rdma(v, True)
                d_cw.wait_recv()
                d_cw.wait_send()
                stage_cw_ref[v] = acc_ref[pl.ds(0, h), col] + comm_cw_ref[v]
                pl.semaphore_signal(credit_cw, inc=1, device_id=(prv,),
                                    device_id_type=pl.DeviceIdType.MESH)
                d_ccw = rs_rdma(v, False)
                d_ccw.wait_recv()
                d_ccw.wait_send()
                stage_ccw_ref[v] = acc_ref[pl.ds(h, h), col] + comm_ccw_ref[v]
                pl.semaphore_signal(credit_ccw, inc=1, device_id=(nxt,),
                                    device_id_type=pl.DeviceIdType.MESH)
                pl.semaphore_wait(credit_cw, 1)
                pl.semaphore_wait(credit_ccw, 1)
                rs_rdma(v, True).start()
                rs_rdma(v, False).start()
            return carry

        lax.fori_loop(1, N_DEV - 1, rs_step, 0)

        acc_ref[:h] = partial_top(q)
        acc_ref[h:] = partial_bot(q)
        for v in range(NSUB):
            col = pl.ds(v * nh, nh)
            d_cw = rs_rdma(v, True)
            d_cw.wait_recv()
            d_cw.wait_send()
            y = (acc_ref[pl.ds(0, h), col] + comm_cw_ref[v]
                 ).astype(jnp.float32)
            stage_cw_ref[v] = (y * jax.nn.sigmoid(y)).astype(jnp.bfloat16)
            d_ccw = rs_rdma(v, False)
            d_ccw.wait_recv()
            d_ccw.wait_send()
            y = (acc_ref[pl.ds(h, h), col] + comm_ccw_ref[v]
                 ).astype(jnp.float32)
            stage_ccw_ref[v] = (y * jax.nn.sigmoid(y)).astype(jnp.bfloat16)

        own = []
        for v in range(NSUB):
            col = pl.ds(v * nh, nh)
            cp_cw = pltpu.make_async_copy(
                stage_cw_ref.at[v], out_ref.at[pl.ds(q * mc, h), col],
                copy_sem)
            cp_cw.start()
            cp_ccw = pltpu.make_async_copy(
                stage_ccw_ref.at[v], out_ref.at[pl.ds(q * mc + h, h), col],
                copy_sem)
            cp_ccw.start()
            own += [cp_cw, cp_ccw]
        for cp in own:
            cp.wait()

        def ag_rdma(t, cw):
            if cw:
                rows = pl.ds(jnp.mod(q - t, N_DEV) * mc, h)
                return pltpu.make_async_remote_copy(
                    src_ref=out_ref.at[rows, :], dst_ref=out_ref.at[rows, :],
                    send_sem=send_sems_cw.at[0], recv_sem=recv_sems_cw.at[0],
                    device_id=(nxt,), device_id_type=pl.DeviceIdType.MESH)
            rows = pl.ds(jnp.mod(q + t, N_DEV) * mc + h, h)
            return pltpu.make_async_remote_copy(
                src_ref=out_ref.at[rows, :], dst_ref=out_ref.at[rows, :],
                send_sem=send_sems_ccw.at[0], recv_sem=recv_sems_ccw.at[0],
                device_id=(prv,), device_id_type=pl.DeviceIdType.MESH)

        ag_rdma(0, True).start()
        ag_rdma(0, False).start()

        def ag_step(t, carry):
            d_cw = ag_rdma(t - 1, True)
            d_cw.wait_recv()
            d_cw.wait_send()
            ag_rdma(t, True).start()
            d_ccw = ag_rdma(t - 1, False)
            d_ccw.wait_recv()
            d_ccw.wait_send()
            ag_rdma(t, False).start()
            return carry

        lax.fori_loop(1, N_DEV - 1, ag_step, 0)

        for cw in (True, False):
            d = ag_rdma(N_DEV - 2, cw)
            d.wait_recv()
            d.wait_send()

    return pl.pallas_call(
        body,
        out_shape=jax.ShapeDtypeStruct((m, n), jnp.bfloat16),
        in_specs=[
            pl.BlockSpec(memory_space=pltpu.VMEM),
            pl.BlockSpec(memory_space=pltpu.VMEM),
        ],
        out_specs=pl.BlockSpec(memory_space=pl.ANY),
        scratch_shapes=[
            pltpu.VMEM((mc, n), jnp.bfloat16),
            pltpu.VMEM((NSUB, h, nh), jnp.bfloat16),
            pltpu.VMEM((NSUB, h, nh), jnp.bfloat16),
            pltpu.VMEM((NSUB, h, nh), jnp.bfloat16),
            pltpu.VMEM((NSUB, h, nh), jnp.bfloat16),
            pltpu.SemaphoreType.DMA((NSUB,)),
            pltpu.SemaphoreType.DMA((NSUB,)),
            pltpu.SemaphoreType.DMA((NSUB,)),
            pltpu.SemaphoreType.DMA((NSUB,)),
            pltpu.SemaphoreType.DMA,
            pltpu.SemaphoreType.REGULAR,
            pltpu.SemaphoreType.REGULAR,
        ],
        compiler_params=pltpu.CompilerParams(
            collective_id=0,
            vmem_limit_bytes=64 * 1024 * 1024,
        ),
    )(x, w_mat)


# device time: 710094 ns/iter; 1.9718x vs baseline; 1.0176x over previous
import jax
import jax.numpy as jnp
from jax import lax
from jax.experimental import pallas as pl
from jax.experimental.pallas import tpu as pltpu

N_DEV = 8
NSUB = 2


def _ring_to_pos(r):
    return jnp.where(r < 4, r, 11 - r)


def kernel(x, w_mat):
    m, k_per = x.shape
    _, n = w_mat.shape
    mc = m // N_DEV
    h = mc // 2
    nh = n // NSUB

    x = x.astype(jnp.bfloat16)
    w_mat = w_mat.astype(jnp.bfloat16)

    def body(x_ref, w_ref, out_ref, acc_ref, stage_cw_ref, stage_ccw_ref,
             comm_cw_ref, comm_ccw_ref,
             send_sems_cw, recv_sems_cw, send_sems_ccw, recv_sems_ccw,
             copy_sem, credit_cw, credit_ccw):
        my = lax.axis_index("i")
        q = _ring_to_pos(my)
        nxt = _ring_to_pos(jnp.mod(q + 1, N_DEV))
        prv = _ring_to_pos(jnp.mod(q - 1, N_DEV))

        barrier = pltpu.get_barrier_semaphore()
        for nbr in (prv, nxt):
            pl.semaphore_signal(barrier, inc=1, device_id=(nbr,),
                                device_id_type=pl.DeviceIdType.MESH)
        pl.semaphore_wait(barrier, 2)

        def partial_top(c):
            return jnp.dot(x_ref[pl.ds(c * mc, h), :], w_ref[...],
                           preferred_element_type=jnp.float32
                           ).astype(jnp.bfloat16)

        def partial_bot(c):
            return jnp.dot(x_ref[pl.ds(c * mc + h, h), :], w_ref[...],
                           preferred_element_type=jnp.float32
                           ).astype(jnp.bfloat16)

        def rs_rdma(v, cw):
            if cw:
                return pltpu.make_async_remote_copy(
                    src_ref=stage_cw_ref.at[v], dst_ref=comm_cw_ref.at[v],
                    send_sem=send_sems_cw.at[v], recv_sem=recv_sems_cw.at[v],
                    device_id=(nxt,), device_id_type=pl.DeviceIdType.MESH)
            return pltpu.make_async_remote_copy(
                src_ref=stage_ccw_ref.at[v], dst_ref=comm_ccw_ref.at[v],
                send_sem=send_sems_ccw.at[v], recv_sem=recv_sems_ccw.at[v],
                device_id=(prv,), device_id_type=pl.DeviceIdType.MESH)

        pt = partial_top(jnp.mod(q - 1, N_DEV))
        stage_cw_ref[0] = pt[:, :nh]
        stage_cw_ref[1] = pt[:, nh:]
        pb = partial_bot(jnp.mod(q + 1, N_DEV))
        stage_ccw_ref[0] = pb[:, :nh]
        stage_ccw_ref[1] = pb[:, nh:]
        for v in range(NSUB):
            rs_rdma(v, True).start()
            rs_rdma(v, False).start()

        def rs_step(s, carry):
            acc_ref[:h] = partial_top(jnp.mod(q - 1 - s, N_DEV))
            acc_ref[h:] = partial_bot(jnp.mod(q + 1 + s, N_DEV))
            for v in range(NSUB):
                col = pl.ds(v * nh, nh)
                d_cw = rs_rdma(v, True)
                d_cw.wait_recv()
                d_cw.wait_send()
                stage_cw_ref[v] = acc_ref[pl.ds(0, h), col] + comm_cw_ref[v]
                pl.semaphore_signal(credit_cw, inc=1, device_id=(prv,),
                                    device_id_type=pl.DeviceIdType.MESH)
                d_ccw = rs_rdma(v, False)
                d_ccw.wait_recv()
                d_ccw.wait_send()
                stage_ccw_ref[v] = acc_ref[pl.ds(h, h), col] + comm_ccw_ref[v]
                pl.semaphore_signal(credit_ccw, inc=1, device_id=(nxt,),
                                    device_id_type=pl.DeviceIdType.MESH)
                pl.semaphore_wait(credit_cw, 1)
                pl.semaphore_wait(credit_ccw, 1)
                rs_rdma(v, True).start()
                rs_rdma(v, False).start()
            return carry

        lax.fori_loop(1, N_DEV - 1, rs_step, 0)

        acc_ref[:h] = partial_top(q)
        acc_ref[h:] = partial_bot(q)
        for v in range(NSUB):
            col = pl.ds(v * nh, nh)
            d_cw = rs_rdma(v, True)
            d_cw.wait_recv()
            d_cw.wait_send()
            y = (acc_ref[pl.ds(0, h), col] + comm_cw_ref[v]
                 ).astype(jnp.float32)
            stage_cw_ref[v] = (y * jax.nn.sigmoid(y)).astype(jnp.bfloat16)
            d_ccw = rs_rdma(v, False)
            d_ccw.wait_recv()
            d_ccw.wait_send()
            y = (acc_ref[pl.ds(h, h), col] + comm_ccw_ref[v]
                 ).astype(jnp.float32)
            stage_ccw_ref[v] = (y * jax.nn.sigmoid(y)).astype(jnp.bfloat16)

        own = []
        for v in range(NSUB):
            col = pl.ds(v * nh, nh)
            cp_cw = pltpu.make_async_copy(
                stage_cw_ref.at[v], out_ref.at[pl.ds(q * mc, h), col],
                copy_sem)
            cp_cw.start()
            cp_ccw = pltpu.make_async_copy(
                stage_ccw_ref.at[v], out_ref.at[pl.ds(q * mc + h, h), col],
                copy_sem)
            cp_ccw.start()
            own += [cp_cw, cp_ccw]
        for cp in own:
            cp.wait()

        hq = h // NSUB

        def ag_rdma(t, cw, r):
            if cw:
                rows = pl.ds(jnp.mod(q - t, N_DEV) * mc + r * hq, hq)
                return pltpu.make_async_remote_copy(
                    src_ref=out_ref.at[rows, :], dst_ref=out_ref.at[rows, :],
                    send_sem=send_sems_cw.at[r], recv_sem=recv_sems_cw.at[r],
                    device_id=(nxt,), device_id_type=pl.DeviceIdType.MESH)
            rows = pl.ds(jnp.mod(q + t, N_DEV) * mc + h + r * hq, hq)
            return pltpu.make_async_remote_copy(
                src_ref=out_ref.at[rows, :], dst_ref=out_ref.at[rows, :],
                send_sem=send_sems_ccw.at[r], recv_sem=recv_sems_ccw.at[r],
                device_id=(prv,), device_id_type=pl.DeviceIdType.MESH)

        for r in range(NSUB):
            ag_rdma(0, True, r).start()
            ag_rdma(0, False, r).start()

        def ag_step(t, carry):
            for r in range(NSUB):
                d_cw = ag_rdma(t - 1, True, r)
                d_cw.wait_recv()
                d_cw.wait_send()
                ag_rdma(t, True, r).start()
                d_ccw = ag_rdma(t - 1, False, r)
                d_ccw.wait_recv()
                d_ccw.wait_send()
                ag_rdma(t, False, r).start()
            return carry

        lax.fori_loop(1, N_DEV - 1, ag_step, 0)

        for cw in (True, False):
            for r in range(NSUB):
                d = ag_rdma(N_DEV - 2, cw, r)
                d.wait_recv()
                d.wait_send()

    return pl.pallas_call(
        body,
        out_shape=jax.ShapeDtypeStruct((m, n), jnp.bfloat16),
        in_specs=[
            pl.BlockSpec(memory_space=pltpu.VMEM),
            pl.BlockSpec(memory_space=pltpu.VMEM),
        ],
        out_specs=pl.BlockSpec(memory_space=pl.ANY),
        scratch_shapes=[
            pltpu.VMEM((mc, n), jnp.bfloat16),
            pltpu.VMEM((NSUB, h, nh), jnp.bfloat16),
            pltpu.VMEM((NSUB, h, nh), jnp.bfloat16),
            pltpu.VMEM((NSUB, h, nh), jnp.bfloat16),
            pltpu.VMEM((NSUB, h, nh), jnp.bfloat16),
            pltpu.SemaphoreType.DMA((NSUB,)),
            pltpu.SemaphoreType.DMA((NSUB,)),
            pltpu.SemaphoreType.DMA((NSUB,)),
            pltpu.SemaphoreType.DMA((NSUB,)),
            pltpu.SemaphoreType.DMA,
            pltpu.SemaphoreType.REGULAR,
            pltpu.SemaphoreType.REGULAR,
        ],
        compiler_params=pltpu.CompilerParams(
            collective_id=0,
            vmem_limit_bytes=64 * 1024 * 1024,
        ),
    )(x, w_mat)


# device time: 706797 ns/iter; 1.9810x vs baseline; 1.0047x over previous
import jax
import jax.numpy as jnp
from jax import lax
from jax.experimental import pallas as pl
from jax.experimental.pallas import tpu as pltpu

N_DEV = 8
NSUB = 2


def _ring_to_pos(r):
    return jnp.where(r < 4, r, 11 - r)


def kernel(x, w_mat):
    m, k_per = x.shape
    _, n = w_mat.shape
    mc = m // N_DEV
    h = mc // 2
    nh = n // NSUB

    x = x.astype(jnp.bfloat16)
    w_mat = w_mat.astype(jnp.bfloat16)

    def body(x_ref, w_ref, out_ref, acc_ref, stage_cw_ref, stage_ccw_ref,
             comm_cw_ref, comm_ccw_ref,
             send_sems_cw, recv_sems_cw, send_sems_ccw, recv_sems_ccw,
             copy_sem, credit_cw, credit_ccw):
        my = lax.axis_index("i")
        q = _ring_to_pos(my)
        nxt = _ring_to_pos(jnp.mod(q + 1, N_DEV))
        prv = _ring_to_pos(jnp.mod(q - 1, N_DEV))

        barrier = pltpu.get_barrier_semaphore()
        for nbr in (prv, nxt):
            pl.semaphore_signal(barrier, inc=1, device_id=(nbr,),
                                device_id_type=pl.DeviceIdType.MESH)
        pl.semaphore_wait(barrier, 2)

        def partial_top(c):
            return jnp.dot(x_ref[pl.ds(c * mc, h), :], w_ref[...],
                           preferred_element_type=jnp.float32
                           ).astype(jnp.bfloat16)

        def partial_bot(c):
            return jnp.dot(x_ref[pl.ds(c * mc + h, h), :], w_ref[...],
                           preferred_element_type=jnp.float32
                           ).astype(jnp.bfloat16)

        def rs_rdma(v, cw):
            if cw:
                return pltpu.make_async_remote_copy(
                    src_ref=stage_cw_ref.at[v], dst_ref=comm_cw_ref.at[v],
                    send_sem=send_sems_cw.at[v], recv_sem=recv_sems_cw.at[v],
                    device_id=(nxt,), device_id_type=pl.DeviceIdType.MESH)
            return pltpu.make_async_remote_copy(
                src_ref=stage_ccw_ref.at[v], dst_ref=comm_ccw_ref.at[v],
                send_sem=send_sems_ccw.at[v], recv_sem=recv_sems_ccw.at[v],
                device_id=(prv,), device_id_type=pl.DeviceIdType.MESH)

        def partial_sub(c, v, top):
            r0 = c * mc if top else c * mc + h
            return jnp.dot(x_ref[pl.ds(r0, h), :],
                           w_ref[:, pl.ds(v * nh, nh)],
                           preferred_element_type=jnp.float32
                           ).astype(jnp.bfloat16)

        c_cw0 = jnp.mod(q - 1, N_DEV)
        c_ccw0 = jnp.mod(q + 1, N_DEV)
        for v in range(NSUB):
            stage_cw_ref[v] = partial_sub(c_cw0, v, True)
            rs_rdma(v, True).start()
            stage_ccw_ref[v] = partial_sub(c_ccw0, v, False)
            rs_rdma(v, False).start()

        def rs_step(s, carry):
            acc_ref[:h] = partial_top(jnp.mod(q - 1 - s, N_DEV))
            acc_ref[h:] = partial_bot(jnp.mod(q + 1 + s, N_DEV))
            for v in range(NSUB):
                col = pl.ds(v * nh, nh)
                d_cw = rs_rdma(v, True)
                d_cw.wait_recv()
                d_cw.wait_send()
                stage_cw_ref[v] = acc_ref[pl.ds(0, h), col] + comm_cw_ref[v]
                pl.semaphore_signal(credit_cw, inc=1, device_id=(prv,),
                                    device_id_type=pl.DeviceIdType.MESH)
                d_ccw = rs_rdma(v, False)
                d_ccw.wait_recv()
                d_ccw.wait_send()
                stage_ccw_ref[v] = acc_ref[pl.ds(h, h), col] + comm_ccw_ref[v]
                pl.semaphore_signal(credit_ccw, inc=1, device_id=(nxt,),
                                    device_id_type=pl.DeviceIdType.MESH)
                pl.semaphore_wait(credit_cw, 1)
                pl.semaphore_wait(credit_ccw, 1)
                rs_rdma(v, True).start()
                rs_rdma(v, False).start()
            return carry

        lax.fori_loop(1, N_DEV - 1, rs_step, 0)

        acc_ref[:h] = partial_top(q)
        acc_ref[h:] = partial_bot(q)
        for v in range(NSUB):
            col = pl.ds(v * nh, nh)
            d_cw = rs_rdma(v, True)
            d_cw.wait_recv()
            d_cw.wait_send()
            y = (acc_ref[pl.ds(0, h), col] + comm_cw_ref[v]
                 ).astype(jnp.float32)
            stage_cw_ref[v] = (y * jax.nn.sigmoid(y)).astype(jnp.bfloat16)
            d_ccw = rs_rdma(v, False)
            d_ccw.wait_recv()
            d_ccw.wait_send()
            y = (acc_ref[pl.ds(h, h), col] + comm_ccw_ref[v]
                 ).astype(jnp.float32)
            stage_ccw_ref[v] = (y * jax.nn.sigmoid(y)).astype(jnp.bfloat16)

        own = []
        for v in range(NSUB):
            col = pl.ds(v * nh, nh)
            cp_cw = pltpu.make_async_copy(
                stage_cw_ref.at[v], out_ref.at[pl.ds(q * mc, h), col],
                copy_sem)
            cp_cw.start()
            cp_ccw = pltpu.make_async_copy(
                stage_ccw_ref.at[v], out_ref.at[pl.ds(q * mc + h, h), col],
                copy_sem)
            cp_ccw.start()
            own += [cp_cw, cp_ccw]
        for cp in own:
            cp.wait()

        hq = h // NSUB

        def ag_rdma(t, cw, r):
            if cw:
                rows = pl.ds(jnp.mod(q - t, N_DEV) * mc + r * hq, hq)
                return pltpu.make_async_remote_copy(
                    src_ref=out_ref.at[rows, :], dst_ref=out_ref.at[rows, :],
                    send_sem=send_sems_cw.at[r], recv_sem=recv_sems_cw.at[r],
                    device_id=(nxt,), device_id_type=pl.DeviceIdType.MESH)
            rows = pl.ds(jnp.mod(q + t, N_DEV) * mc + h + r * hq, hq)
            return pltpu.make_async_remote_copy(
                src_ref=out_ref.at[rows, :], dst_ref=out_ref.at[rows, :],
                send_sem=send_sems_ccw.at[r], recv_sem=recv_sems_ccw.at[r],
                device_id=(prv,), device_id_type=pl.DeviceIdType.MESH)

        for r in range(NSUB):
            ag_rdma(0, True, r).start()
            ag_rdma(0, False, r).start()

        def ag_step(t, carry):
            for r in range(NSUB):
                d_cw = ag_rdma(t - 1, True, r)
                d_cw.wait_recv()
                d_cw.wait_send()
                ag_rdma(t, True, r).start()
                d_ccw = ag_rdma(t - 1, False, r)
                d_ccw.wait_recv()
                d_ccw.wait_send()
                ag_rdma(t, False, r).start()
            return carry

        lax.fori_loop(1, N_DEV - 1, ag_step, 0)

        for cw in (True, False):
            for r in range(NSUB):
                d = ag_rdma(N_DEV - 2, cw, r)
                d.wait_recv()
                d.wait_send()

    return pl.pallas_call(
        body,
        out_shape=jax.ShapeDtypeStruct((m, n), jnp.bfloat16),
        in_specs=[
            pl.BlockSpec(memory_space=pltpu.VMEM),
            pl.BlockSpec(memory_space=pltpu.VMEM),
        ],
        out_specs=pl.BlockSpec(memory_space=pl.ANY),
        scratch_shapes=[
            pltpu.VMEM((mc, n), jnp.bfloat16),
            pltpu.VMEM((NSUB, h, nh), jnp.bfloat16),
            pltpu.VMEM((NSUB, h, nh), jnp.bfloat16),
            pltpu.VMEM((NSUB, h, nh), jnp.bfloat16),
            pltpu.VMEM((NSUB, h, nh), jnp.bfloat16),
            pltpu.SemaphoreType.DMA((NSUB,)),
            pltpu.SemaphoreType.DMA((NSUB,)),
            pltpu.SemaphoreType.DMA((NSUB,)),
            pltpu.SemaphoreType.DMA((NSUB,)),
            pltpu.SemaphoreType.DMA,
            pltpu.SemaphoreType.REGULAR,
            pltpu.SemaphoreType.REGULAR,
        ],
        compiler_params=pltpu.CompilerParams(
            collective_id=0,
            vmem_limit_bytes=64 * 1024 * 1024,
        ),
    )(x, w_mat)


# device time: 704903 ns/iter; 1.9863x vs baseline; 1.0027x over previous
import jax
import jax.numpy as jnp
from jax import lax
from jax.experimental import pallas as pl
from jax.experimental.pallas import tpu as pltpu

N_DEV = 8
NSUB = 2


def _ring_to_pos(r):
    return jnp.where(r < 4, r, 11 - r)


def kernel(x, w_mat):
    m, k_per = x.shape
    _, n = w_mat.shape
    mc = m // N_DEV
    h = mc // 2
    nh = n // NSUB

    x = x.astype(jnp.bfloat16)
    w_mat = w_mat.astype(jnp.bfloat16)

    def body(x_ref, w_ref, out_ref, acc_ref, stage_cw_ref, stage_ccw_ref,
             comm_cw_ref, comm_ccw_ref,
             send_sems_cw, recv_sems_cw, send_sems_ccw, recv_sems_ccw,
             copy_sem, credit_cw, credit_ccw):
        my = lax.axis_index("i")
        q = _ring_to_pos(my)
        nxt = _ring_to_pos(jnp.mod(q + 1, N_DEV))
        prv = _ring_to_pos(jnp.mod(q - 1, N_DEV))

        barrier = pltpu.get_barrier_semaphore()
        for nbr in (prv, nxt):
            pl.semaphore_signal(barrier, inc=1, device_id=(nbr,),
                                device_id_type=pl.DeviceIdType.MESH)
        pl.semaphore_wait(barrier, 2)

        def partial_top(c):
            return jnp.dot(x_ref[pl.ds(c * mc, h), :], w_ref[...],
                           preferred_element_type=jnp.float32
                           ).astype(jnp.bfloat16)

        def partial_bot(c):
            return jnp.dot(x_ref[pl.ds(c * mc + h, h), :], w_ref[...],
                           preferred_element_type=jnp.float32
                           ).astype(jnp.bfloat16)

        def rs_rdma(v, cw):
            if cw:
                return pltpu.make_async_remote_copy(
                    src_ref=stage_cw_ref.at[v], dst_ref=comm_cw_ref.at[v],
                    send_sem=send_sems_cw.at[v], recv_sem=recv_sems_cw.at[v],
                    device_id=(nxt,), device_id_type=pl.DeviceIdType.MESH)
            return pltpu.make_async_remote_copy(
                src_ref=stage_ccw_ref.at[v], dst_ref=comm_ccw_ref.at[v],
                send_sem=send_sems_ccw.at[v], recv_sem=recv_sems_ccw.at[v],
                device_id=(prv,), device_id_type=pl.DeviceIdType.MESH)

        def partial_sub(c, v, top):
            r0 = c * mc if top else c * mc + h
            return jnp.dot(x_ref[pl.ds(r0, h), :],
                           w_ref[:, pl.ds(v * nh, nh)],
                           preferred_element_type=jnp.float32
                           ).astype(jnp.bfloat16)

        c_cw0 = jnp.mod(q - 1, N_DEV)
        c_ccw0 = jnp.mod(q + 1, N_DEV)
        for v in range(NSUB):
            stage_cw_ref[v] = partial_sub(c_cw0, v, True)
            rs_rdma(v, True).start()
            stage_ccw_ref[v] = partial_sub(c_ccw0, v, False)
            rs_rdma(v, False).start()

        def rs_step(s, carry):
            acc_ref[:h] = partial_top(jnp.mod(q - 1 - s, N_DEV))
            acc_ref[h:] = partial_bot(jnp.mod(q + 1 + s, N_DEV))
            for v in range(NSUB):
                col = pl.ds(v * nh, nh)
                d_cw = rs_rdma(v, True)
                d_cw.wait_recv()
                d_cw.wait_send()
                stage_cw_ref[v] = acc_ref[pl.ds(0, h), col] + comm_cw_ref[v]
                pl.semaphore_signal(credit_cw, inc=1, device_id=(prv,),
                                    device_id_type=pl.DeviceIdType.MESH)
                d_ccw = rs_rdma(v, False)
                d_ccw.wait_recv()
                d_ccw.wait_send()
                stage_ccw_ref[v] = acc_ref[pl.ds(h, h), col] + comm_ccw_ref[v]
                pl.semaphore_signal(credit_ccw, inc=1, device_id=(nxt,),
                                    device_id_type=pl.DeviceIdType.MESH)
                pl.semaphore_wait(credit_cw, 1)
                pl.semaphore_wait(credit_ccw, 1)
                rs_rdma(v, True).start()
                rs_rdma(v, False).start()
            return carry

        lax.fori_loop(1, N_DEV - 1, rs_step, 0)

        acc_ref[:h] = partial_top(q)
        acc_ref[h:] = partial_bot(q)
        own = []
        for v in range(NSUB):
            col = pl.ds(v * nh, nh)
            d_cw = rs_rdma(v, True)
            d_cw.wait_recv()
            d_cw.wait_send()
            y = (acc_ref[pl.ds(0, h), col] + comm_cw_ref[v]
                 ).astype(jnp.float32)
            stage_cw_ref[v] = (y * jax.nn.sigmoid(y)).astype(jnp.bfloat16)
            cp_cw = pltpu.make_async_copy(
                stage_cw_ref.at[v], out_ref.at[pl.ds(q * mc, h), col],
                copy_sem)
            cp_cw.start()
            d_ccw = rs_rdma(v, False)
            d_ccw.wait_recv()
            d_ccw.wait_send()
            y = (acc_ref[pl.ds(h, h), col] + comm_ccw_ref[v]
                 ).astype(jnp.float32)
            stage_ccw_ref[v] = (y * jax.nn.sigmoid(y)).astype(jnp.bfloat16)
            cp_ccw = pltpu.make_async_copy(
                stage_ccw_ref.at[v], out_ref.at[pl.ds(q * mc + h, h), col],
                copy_sem)
            cp_ccw.start()
            own += [cp_cw, cp_ccw]
        for cp in own:
            cp.wait()

        hq = h // NSUB

        def ag_rdma(t, cw, r):
            if cw:
                rows = pl.ds(jnp.mod(q - t, N_DEV) * mc + r * hq, hq)
                return pltpu.make_async_remote_copy(
                    src_ref=out_ref.at[rows, :], dst_ref=out_ref.at[rows, :],
                    send_sem=send_sems_cw.at[r], recv_sem=recv_sems_cw.at[r],
                    device_id=(nxt,), device_id_type=pl.DeviceIdType.MESH)
            rows = pl.ds(jnp.mod(q + t, N_DEV) * mc + h + r * hq, hq)
            return pltpu.make_async_remote_copy(
                src_ref=out_ref.at[rows, :], dst_ref=out_ref.at[rows, :],
                send_sem=send_sems_ccw.at[r], recv_sem=recv_sems_ccw.at[r],
                device_id=(prv,), device_id_type=pl.DeviceIdType.MESH)

        for r in range(NSUB):
            ag_rdma(0, True, r).start()
            ag_rdma(0, False, r).start()

        def ag_step(t, carry):
            for r in range(NSUB):
                d_cw = ag_rdma(t - 1, True, r)
                d_cw.wait_recv()
                d_cw.wait_send()
                ag_rdma(t, True, r).start()
                d_ccw = ag_rdma(t - 1, False, r)
                d_ccw.wait_recv()
                d_ccw.wait_send()
                ag_rdma(t, False, r).start()
            return carry

        lax.fori_loop(1, N_DEV - 1, ag_step, 0)

        for cw in (True, False):
            for r in range(NSUB):
                d = ag_rdma(N_DEV - 2, cw, r)
                d.wait_recv()
                d.wait_send()

    return pl.pallas_call(
        body,
        out_shape=jax.ShapeDtypeStruct((m, n), jnp.bfloat16),
        in_specs=[
            pl.BlockSpec(memory_space=pltpu.VMEM),
            pl.BlockSpec(memory_space=pltpu.VMEM),
        ],
        out_specs=pl.BlockSpec(memory_space=pl.ANY),
        scratch_shapes=[
            pltpu.VMEM((mc, n), jnp.bfloat16),
            pltpu.VMEM((NSUB, h, nh), jnp.bfloat16),
            pltpu.VMEM((NSUB, h, nh), jnp.bfloat16),
            pltpu.VMEM((NSUB, h, nh), jnp.bfloat16),
            pltpu.VMEM((NSUB, h, nh), jnp.bfloat16),
            pltpu.SemaphoreType.DMA((NSUB,)),
            pltpu.SemaphoreType.DMA((NSUB,)),
            pltpu.SemaphoreType.DMA((NSUB,)),
            pltpu.SemaphoreType.DMA((NSUB,)),
            pltpu.SemaphoreType.DMA,
            pltpu.SemaphoreType.REGULAR,
            pltpu.SemaphoreType.REGULAR,
        ],
        compiler_params=pltpu.CompilerParams(
            collective_id=0,
            vmem_limit_bytes=64 * 1024 * 1024,
        ),
    )(x, w_mat)
